# baseline (speedup 1.0000x reference)
import functools

import jax
import jax.numpy as jnp
from jax import lax
from jax.experimental import pallas as pl
from jax.experimental.pallas import tpu as pltpu
from jax.experimental.pallas import tpu_sc as plsc

NC = 2
NS = 16
NW = NC * NS
L = 16
CHUNK = 8192
TAIL_PAD = 512
UNROLL = 8
RC_BITS = 14
RC_MASK = (1 << RC_BITS) - 1


def _sc_partials(vals, rc, tails, x_flat):
    m = x_flat.shape[0]
    nnz = vals.shape[0]
    share = (nnz // (NW * L)) * L
    n_full = share // CHUNK
    tail = share - n_full * CHUNK
    chunks = [(i * CHUNK, CHUNK) for i in range(n_full)]
    if tail:
        chunks.append((n_full * CHUNK, tail))
    nchunks = len(chunks)

    mesh = plsc.VectorSubcoreMesh(core_axis_name="c", subcore_axis_name="s")

    @functools.partial(
        pl.kernel,
        out_type=jax.ShapeDtypeStruct((NW, m), jnp.float32),
        mesh=mesh,
        compiler_params=pltpu.CompilerParams(needs_layout_passes=False),
        scratch_types=[
            pltpu.VMEM((m,), jnp.float32),
            pltpu.VMEM((m,), jnp.float32),
            pltpu.VMEM((CHUNK,), jnp.float32),
            pltpu.VMEM((CHUNK,), jnp.int32),
            pltpu.VMEM((CHUNK,), jnp.float32),
            pltpu.VMEM((CHUNK,), jnp.int32),
            pltpu.VMEM((TAIL_PAD,), jnp.int32),
            pltpu.SemaphoreType.DMA,
            pltpu.SemaphoreType.DMA,
        ],
    )
    def k(vals_hbm, rc_hbm, tails_hbm, x_hbm, out_hbm,
          x_v, acc_v, va, rca, vb, rcb, tvb, sa, sb):
        wid = lax.axis_index("s") * NC + lax.axis_index("c")
        base = wid * share
        bufs = [(va, rca, sa), (vb, rcb, sb)]

        pltpu.sync_copy(x_hbm, x_v)

        zero = jnp.zeros((L,), jnp.float32)

        def zbody(g, carry):
            for t in range(UNROLL):
                acc_v[pl.ds((g * UNROLL + t) * L, L)] = zero
            return carry

        lax.fori_loop(0, m // (L * UNROLL), zbody, 0)

        descs = {}

        def start(ci):
            off, sz = chunks[ci]
            vv, rcv, sem = bufs[ci % 2]
            s = pl.ds(base + off, sz)
            d = pl.ds(0, sz)
            descs[ci] = (
                pltpu.async_copy(vals_hbm.at[s], vv.at[d], sem),
                pltpu.async_copy(rc_hbm.at[s], rcv.at[d], sem),
            )

        def step(vv, rcv, j):
            s = pl.ds(j * L, L)
            rcx = rcv[s]
            cols = lax.shift_right_logical(rcx, RC_BITS)
            rows = jnp.bitwise_and(rcx, RC_MASK)
            xg = plsc.load_gather(x_v, [cols])
            plsc.addupdate_scatter(acc_v, [rows], vv[s] * xg)

        def compute(vv, rcv, sz, unroll):
            def vbody(g, carry):
                for t in range(unroll):
                    step(vv, rcv, g * unroll + t)
                return carry

            lax.fori_loop(0, sz // (L * unroll), vbody, 0)
            for j in range(sz // (L * unroll) * unroll, sz // L):
                step(vv, rcv, j)

        start(0)
        if nchunks > 1:
            start(1)
        for ci in range(nchunks):
            for dsc in descs.pop(ci):
                dsc.wait()
            off, sz = chunks[ci]
            vv, rcv, _ = bufs[ci % 2]
            compute(vv, rcv, sz, UNROLL)
            if ci + 2 < nchunks:
                start(ci + 2)

        @pl.when(wid == 0)
        def _():
            d = pl.ds(0, TAIL_PAD)
            pltpu.sync_copy(tails_hbm.at[pl.ds(0, TAIL_PAD)], tvb)
            pltpu.sync_copy(tails_hbm.at[pl.ds(TAIL_PAD, TAIL_PAD)],
                            rca.at[d])

            def tbody(g, carry):
                for t in range(UNROLL):
                    j = g * UNROLL + t
                    s = pl.ds(j * L, L)
                    rcx = rca[s]
                    cols = lax.shift_right_logical(rcx, RC_BITS)
                    rows = jnp.bitwise_and(rcx, RC_MASK)
                    xg = plsc.load_gather(x_v, [cols])
                    vv = plsc.bitcast(tvb[s], jnp.float32)
                    plsc.addupdate_scatter(acc_v, [rows], vv * xg)
                return carry

            lax.fori_loop(0, TAIL_PAD // (L * UNROLL), tbody, 0)

        pltpu.sync_copy(acc_v, out_hbm.at[wid])

    return k(vals, rc, tails, x_flat)


def _finish(partials, b2, x2, Iy2, il2, iu2, l2, u2):
    def body(p_ref, b_ref, x_ref, iy_ref, il_ref, iu_ref, l_ref, u_ref,
             o_ref):
        ax = jnp.sum(p_ref[...], axis=0, keepdims=True)
        cons = ax - b_ref[...]
        cons = cons + jnp.maximum(-cons, 0.0) * iy_ref[...]
        xv = x_ref[...]
        var = (jnp.maximum(l_ref[...] - xv, 0.0) * il_ref[...]
               + jnp.maximum(xv - u_ref[...], 0.0) * iu_ref[...])
        part2 = jnp.maximum(jnp.max(jnp.abs(cons)), jnp.max(jnp.abs(var)))
        part3 = 1.0 + jnp.max(jnp.abs(b_ref[...]))
        o_ref[0, 0] = part2 / part3

    return pl.pallas_call(
        body,
        out_shape=jax.ShapeDtypeStruct((1, 1), jnp.float32),
        out_specs=pl.BlockSpec(memory_space=pltpu.SMEM),
    )(partials, b2, x2, Iy2, il2, iu2, l2, u2)


def kernel(A_vals, b, c, x, Iy, il, iu, l, u, A_rows, A_cols):
    nnz = A_vals.shape[0]
    n = x.shape[0]
    covered = (nnz // (NW * L)) * L * NW
    rc = (A_rows.astype(jnp.int32)
          | (A_cols.astype(jnp.int32) << RC_BITS))
    tails = jnp.pad(
        jnp.stack([
            lax.bitcast_convert_type(A_vals[covered:], jnp.int32),
            rc[covered:],
        ]),
        ((0, 0), (0, TAIL_PAD - (nnz - covered))),
    ).reshape(-1)

    partials = _sc_partials(A_vals, rc, tails, x[:, 0])

    r = lambda a: a.reshape(1, n)
    out = _finish(partials, b.reshape(1, -1), r(x), r(Iy), r(il), r(iu),
                  r(l), r(u))
    return out[0, 0]


# speedup vs baseline: 1.9539x; 1.9539x over previous
import functools

import jax
import jax.numpy as jnp
from jax import lax
from jax.experimental import pallas as pl
from jax.experimental.pallas import tpu as pltpu
from jax.experimental.pallas import tpu_sc as plsc

NC = 2
NS = 16
NW = NC * NS
L = 16
CHUNK = 8192
TAIL_PAD = 512
UNROLL = 8


def _sc_partials(vals, rows, cols, tails, x_flat):
    m = x_flat.shape[0]
    nnz = vals.shape[0]
    share = (nnz // (NW * L)) * L
    n_full = share // CHUNK
    tail = share - n_full * CHUNK
    chunks = [(i * CHUNK, CHUNK) for i in range(n_full)]
    if tail:
        chunks.append((n_full * CHUNK, tail))
    nchunks = len(chunks)

    mesh = plsc.VectorSubcoreMesh(core_axis_name="c", subcore_axis_name="s")

    @functools.partial(
        pl.kernel,
        out_type=jax.ShapeDtypeStruct((NW, m), jnp.float32),
        mesh=mesh,
        compiler_params=pltpu.CompilerParams(needs_layout_passes=False),
        scratch_types=[
            pltpu.VMEM((m,), jnp.float32),
            pltpu.VMEM((m,), jnp.float32),
            pltpu.VMEM((CHUNK,), jnp.float32),
            pltpu.VMEM((CHUNK,), jnp.int32),
            pltpu.VMEM((CHUNK,), jnp.int32),
            pltpu.VMEM((CHUNK,), jnp.float32),
            pltpu.VMEM((CHUNK,), jnp.int32),
            pltpu.VMEM((CHUNK,), jnp.int32),
            pltpu.VMEM((TAIL_PAD,), jnp.int32),
            pltpu.SemaphoreType.DMA,
            pltpu.SemaphoreType.DMA,
        ],
    )
    def k(vals_hbm, rows_hbm, cols_hbm, tails_hbm, x_hbm, out_hbm,
          x_v, acc_v, va, ra, ca, vb, rb, cb, tvb, sa, sb):
        wid = lax.axis_index("s") * NC + lax.axis_index("c")
        base = wid * share
        bufs = [(va, ra, ca, sa), (vb, rb, cb, sb)]

        descs = {}

        def start(ci):
            off, sz = chunks[ci]
            vv, rv, cv, sem = bufs[ci % 2]
            s = pl.ds(base + off, sz)
            d = pl.ds(0, sz)
            descs[ci] = (
                pltpu.async_copy(vals_hbm.at[s], vv.at[d], sem),
                pltpu.async_copy(rows_hbm.at[s], rv.at[d], sem),
                pltpu.async_copy(cols_hbm.at[s], cv.at[d], sem),
            )

        start(0)
        if nchunks > 1:
            start(1)

        pltpu.sync_copy(x_hbm, x_v)

        zero = jnp.zeros((L,), jnp.float32)

        def zbody(g, carry):
            for t in range(UNROLL):
                acc_v[pl.ds((g * UNROLL + t) * L, L)] = zero
            return carry

        lax.fori_loop(0, m // (L * UNROLL), zbody, 0)

        def steps(vv, rv, cv, js):
            prods = []
            for j in js:
                s = pl.ds(j * L, L)
                xg = plsc.load_gather(x_v, [cv[s]])
                prods.append((rv[s], vv[s] * xg))
            for rx, px in prods:
                plsc.addupdate_scatter(acc_v, [rx], px)

        def compute(vv, rv, cv, sz, unroll):
            def vbody(g, carry):
                steps(vv, rv, cv, [g * unroll + t for t in range(unroll)])
                return carry

            lax.fori_loop(0, sz // (L * unroll), vbody, 0)
            rem = range(sz // (L * unroll) * unroll, sz // L)
            if len(rem):
                steps(vv, rv, cv, list(rem))

        for ci in range(nchunks):
            for dsc in descs.pop(ci):
                dsc.wait()
            off, sz = chunks[ci]
            vv, rv, cv, _ = bufs[ci % 2]
            compute(vv, rv, cv, sz, UNROLL)
            if ci + 2 < nchunks:
                start(ci + 2)

        @pl.when(wid == 0)
        def _():
            d = pl.ds(0, TAIL_PAD)
            pltpu.sync_copy(tails_hbm.at[pl.ds(0, TAIL_PAD)], tvb)
            pltpu.sync_copy(tails_hbm.at[pl.ds(TAIL_PAD, TAIL_PAD)],
                            ra.at[d])
            pltpu.sync_copy(tails_hbm.at[pl.ds(2 * TAIL_PAD, TAIL_PAD)],
                            ca.at[d])

            def tbody(g, carry):
                prods = []
                for t in range(UNROLL):
                    j = g * UNROLL + t
                    s = pl.ds(j * L, L)
                    xg = plsc.load_gather(x_v, [ca[s]])
                    vv = plsc.bitcast(tvb[s], jnp.float32)
                    prods.append((ra[s], vv * xg))
                for rx, px in prods:
                    plsc.addupdate_scatter(acc_v, [rx], px)
                return carry

            lax.fori_loop(0, TAIL_PAD // (L * UNROLL), tbody, 0)

        pltpu.sync_copy(acc_v, out_hbm.at[wid])

    return k(vals, rows, cols, tails, x_flat)


def _finish(partials, b2, x2, Iy2, il2, iu2, l2, u2):
    def body(p_ref, b_ref, x_ref, iy_ref, il_ref, iu_ref, l_ref, u_ref,
             o_ref):
        ax = jnp.sum(p_ref[...], axis=0, keepdims=True)
        cons = ax - b_ref[...]
        cons = cons + jnp.maximum(-cons, 0.0) * iy_ref[...]
        xv = x_ref[...]
        var = (jnp.maximum(l_ref[...] - xv, 0.0) * il_ref[...]
               + jnp.maximum(xv - u_ref[...], 0.0) * iu_ref[...])
        part2 = jnp.maximum(jnp.max(jnp.abs(cons)), jnp.max(jnp.abs(var)))
        part3 = 1.0 + jnp.max(jnp.abs(b_ref[...]))
        o_ref[0, 0] = part2 / part3

    return pl.pallas_call(
        body,
        out_shape=jax.ShapeDtypeStruct((1, 1), jnp.float32),
        out_specs=pl.BlockSpec(memory_space=pltpu.SMEM),
    )(partials, b2, x2, Iy2, il2, iu2, l2, u2)


def kernel(A_vals, b, c, x, Iy, il, iu, l, u, A_rows, A_cols):
    nnz = A_vals.shape[0]
    n = x.shape[0]
    covered = (nnz // (NW * L)) * L * NW
    rows32 = A_rows.astype(jnp.int32)
    cols32 = A_cols.astype(jnp.int32)
    tails = jnp.pad(
        jnp.stack([
            lax.bitcast_convert_type(A_vals[covered:], jnp.int32),
            rows32[covered:],
            cols32[covered:],
        ]),
        ((0, 0), (0, TAIL_PAD - (nnz - covered))),
    ).reshape(-1)

    partials = _sc_partials(A_vals, rows32, cols32, tails, x[:, 0])

    r = lambda a: a.reshape(1, n)
    out = _finish(partials, b.reshape(1, -1), r(x), r(Iy), r(il), r(iu),
                  r(l), r(u))
    return out[0, 0]


# speedup vs baseline: 1.9605x; 1.0034x over previous
import functools

import jax
import jax.numpy as jnp
from jax import lax
from jax.experimental import pallas as pl
from jax.experimental.pallas import tpu as pltpu
from jax.experimental.pallas import tpu_sc as plsc

NC = 2
NS = 16
NW = NC * NS
L = 16
CHUNK = 8192
TAIL_PAD = 512
UNROLL = 16


def _sc_partials(vals, rows, cols, tails, x_flat):
    m = x_flat.shape[0]
    nnz = vals.shape[0]
    share = (nnz // (NW * L)) * L
    n_full = share // CHUNK
    tail = share - n_full * CHUNK
    chunks = [(i * CHUNK, CHUNK) for i in range(n_full)]
    if tail:
        chunks.append((n_full * CHUNK, tail))
    nchunks = len(chunks)

    mesh = plsc.VectorSubcoreMesh(core_axis_name="c", subcore_axis_name="s")

    @functools.partial(
        pl.kernel,
        out_type=jax.ShapeDtypeStruct((NW, m), jnp.float32),
        mesh=mesh,
        compiler_params=pltpu.CompilerParams(needs_layout_passes=False),
        scratch_types=[
            pltpu.VMEM((m,), jnp.float32),
            pltpu.VMEM((m,), jnp.float32),
            pltpu.VMEM((CHUNK,), jnp.float32),
            pltpu.VMEM((CHUNK,), jnp.int32),
            pltpu.VMEM((CHUNK,), jnp.int32),
            pltpu.VMEM((CHUNK,), jnp.float32),
            pltpu.VMEM((CHUNK,), jnp.int32),
            pltpu.VMEM((CHUNK,), jnp.int32),
            pltpu.VMEM((TAIL_PAD,), jnp.int32),
            pltpu.SemaphoreType.DMA,
            pltpu.SemaphoreType.DMA,
        ],
    )
    def k(vals_hbm, rows_hbm, cols_hbm, tails_hbm, x_hbm, out_hbm,
          x_v, acc_v, va, ra, ca, vb, rb, cb, tvb, sa, sb):
        wid = lax.axis_index("s") * NC + lax.axis_index("c")
        base = wid * share
        bufs = [(va, ra, ca, sa), (vb, rb, cb, sb)]

        descs = {}

        def start(ci):
            off, sz = chunks[ci]
            vv, rv, cv, sem = bufs[ci % 2]
            s = pl.ds(base + off, sz)
            d = pl.ds(0, sz)
            descs[ci] = (
                pltpu.async_copy(vals_hbm.at[s], vv.at[d], sem),
                pltpu.async_copy(rows_hbm.at[s], rv.at[d], sem),
                pltpu.async_copy(cols_hbm.at[s], cv.at[d], sem),
            )

        start(0)
        if nchunks > 1:
            start(1)

        pltpu.sync_copy(x_hbm, x_v)

        zero = jnp.zeros((L,), jnp.float32)

        def zbody(g, carry):
            for t in range(UNROLL):
                acc_v[pl.ds((g * UNROLL + t) * L, L)] = zero
            return carry

        lax.fori_loop(0, m // (L * UNROLL), zbody, 0)

        def steps(vv, rv, cv, js):
            prods = []
            for j in js:
                s = pl.ds(j * L, L)
                xg = plsc.load_gather(x_v, [cv[s]])
                prods.append((rv[s], vv[s] * xg))
            for rx, px in prods:
                plsc.addupdate_scatter(acc_v, [rx], px)

        def compute(vv, rv, cv, sz, unroll):
            def vbody(g, carry):
                steps(vv, rv, cv, [g * unroll + t for t in range(unroll)])
                return carry

            lax.fori_loop(0, sz // (L * unroll), vbody, 0)
            rem = range(sz // (L * unroll) * unroll, sz // L)
            if len(rem):
                steps(vv, rv, cv, list(rem))

        for ci in range(nchunks):
            for dsc in descs.pop(ci):
                dsc.wait()
            off, sz = chunks[ci]
            vv, rv, cv, _ = bufs[ci % 2]
            compute(vv, rv, cv, sz, UNROLL)
            if ci + 2 < nchunks:
                start(ci + 2)

        @pl.when(wid == 0)
        def _():
            d = pl.ds(0, TAIL_PAD)
            pltpu.sync_copy(tails_hbm.at[pl.ds(0, TAIL_PAD)], tvb)
            pltpu.sync_copy(tails_hbm.at[pl.ds(TAIL_PAD, TAIL_PAD)],
                            ra.at[d])
            pltpu.sync_copy(tails_hbm.at[pl.ds(2 * TAIL_PAD, TAIL_PAD)],
                            ca.at[d])

            def tbody(g, carry):
                prods = []
                for t in range(UNROLL):
                    j = g * UNROLL + t
                    s = pl.ds(j * L, L)
                    xg = plsc.load_gather(x_v, [ca[s]])
                    vv = plsc.bitcast(tvb[s], jnp.float32)
                    prods.append((ra[s], vv * xg))
                for rx, px in prods:
                    plsc.addupdate_scatter(acc_v, [rx], px)
                return carry

            lax.fori_loop(0, TAIL_PAD // (L * UNROLL), tbody, 0)

        pltpu.sync_copy(acc_v, out_hbm.at[wid])

    return k(vals, rows, cols, tails, x_flat)


def _finish(partials, b2, x2, Iy2, il2, iu2, l2, u2):
    def body(p_ref, b_ref, x_ref, iy_ref, il_ref, iu_ref, l_ref, u_ref,
             o_ref):
        ax = jnp.sum(p_ref[...], axis=0, keepdims=True)
        cons = ax - b_ref[...]
        cons = cons + jnp.maximum(-cons, 0.0) * iy_ref[...]
        xv = x_ref[...]
        var = (jnp.maximum(l_ref[...] - xv, 0.0) * il_ref[...]
               + jnp.maximum(xv - u_ref[...], 0.0) * iu_ref[...])
        part2 = jnp.maximum(jnp.max(jnp.abs(cons)), jnp.max(jnp.abs(var)))
        part3 = 1.0 + jnp.max(jnp.abs(b_ref[...]))
        o_ref[0, 0] = part2 / part3

    return pl.pallas_call(
        body,
        out_shape=jax.ShapeDtypeStruct((1, 1), jnp.float32),
        out_specs=pl.BlockSpec(memory_space=pltpu.SMEM),
    )(partials, b2, x2, Iy2, il2, iu2, l2, u2)


def kernel(A_vals, b, c, x, Iy, il, iu, l, u, A_rows, A_cols):
    nnz = A_vals.shape[0]
    n = x.shape[0]
    covered = (nnz // (NW * L)) * L * NW
    rows32 = A_rows.astype(jnp.int32)
    cols32 = A_cols.astype(jnp.int32)
    tails = jnp.pad(
        jnp.stack([
            lax.bitcast_convert_type(A_vals[covered:], jnp.int32),
            rows32[covered:],
            cols32[covered:],
        ]),
        ((0, 0), (0, TAIL_PAD - (nnz - covered))),
    ).reshape(-1)

    partials = _sc_partials(A_vals, rows32, cols32, tails, x[:, 0])

    r = lambda a: a.reshape(1, n)
    out = _finish(partials, b.reshape(1, -1), r(x), r(Iy), r(il), r(iu),
                  r(l), r(u))
    return out[0, 0]


# speedup vs baseline: 1.9707x; 1.0052x over previous
import functools

import jax
import jax.numpy as jnp
from jax import lax
from jax.experimental import pallas as pl
from jax.experimental.pallas import tpu as pltpu
from jax.experimental.pallas import tpu_sc as plsc

NC = 2
NS = 16
NW = NC * NS
L = 16
CHUNK = 12288
TAIL_PAD = 512
UNROLL = 16


def _sc_partials(vals, rows, cols, tails, x_flat):
    m = x_flat.shape[0]
    nnz = vals.shape[0]
    share = (nnz // (NW * L)) * L
    n_full = share // CHUNK
    tail = share - n_full * CHUNK
    chunks = [(i * CHUNK, CHUNK) for i in range(n_full)]
    if tail:
        chunks.append((n_full * CHUNK, tail))
    nchunks = len(chunks)

    mesh = plsc.VectorSubcoreMesh(core_axis_name="c", subcore_axis_name="s")

    @functools.partial(
        pl.kernel,
        out_type=jax.ShapeDtypeStruct((NW, m), jnp.float32),
        mesh=mesh,
        compiler_params=pltpu.CompilerParams(needs_layout_passes=False),
        scratch_types=[
            pltpu.VMEM((m,), jnp.float32),
            pltpu.VMEM((m,), jnp.float32),
            pltpu.VMEM((CHUNK,), jnp.float32),
            pltpu.VMEM((CHUNK,), jnp.int32),
            pltpu.VMEM((CHUNK,), jnp.int32),
            pltpu.VMEM((CHUNK,), jnp.float32),
            pltpu.VMEM((CHUNK,), jnp.int32),
            pltpu.VMEM((CHUNK,), jnp.int32),
            pltpu.VMEM((TAIL_PAD,), jnp.int32),
            pltpu.SemaphoreType.DMA,
            pltpu.SemaphoreType.DMA,
        ],
    )
    def k(vals_hbm, rows_hbm, cols_hbm, tails_hbm, x_hbm, out_hbm,
          x_v, acc_v, va, ra, ca, vb, rb, cb, tvb, sa, sb):
        wid = lax.axis_index("s") * NC + lax.axis_index("c")
        base = wid * share
        bufs = [(va, ra, ca, sa), (vb, rb, cb, sb)]

        descs = {}

        def start(ci):
            off, sz = chunks[ci]
            vv, rv, cv, sem = bufs[ci % 2]
            s = pl.ds(base + off, sz)
            d = pl.ds(0, sz)
            descs[ci] = (
                pltpu.async_copy(vals_hbm.at[s], vv.at[d], sem),
                pltpu.async_copy(rows_hbm.at[s], rv.at[d], sem),
                pltpu.async_copy(cols_hbm.at[s], cv.at[d], sem),
            )

        start(0)
        if nchunks > 1:
            start(1)

        pltpu.sync_copy(x_hbm, x_v)

        zero = jnp.zeros((L,), jnp.float32)

        def zbody(g, carry):
            for t in range(UNROLL):
                acc_v[pl.ds((g * UNROLL + t) * L, L)] = zero
            return carry

        lax.fori_loop(0, m // (L * UNROLL), zbody, 0)

        def steps(vv, rv, cv, js):
            prods = []
            for j in js:
                s = pl.ds(j * L, L)
                xg = plsc.load_gather(x_v, [cv[s]])
                prods.append((rv[s], vv[s] * xg))
            for rx, px in prods:
                plsc.addupdate_scatter(acc_v, [rx], px)

        def compute(vv, rv, cv, sz, unroll):
            def vbody(g, carry):
                steps(vv, rv, cv, [g * unroll + t for t in range(unroll)])
                return carry

            lax.fori_loop(0, sz // (L * unroll), vbody, 0)
            rem = range(sz // (L * unroll) * unroll, sz // L)
            if len(rem):
                steps(vv, rv, cv, list(rem))

        for ci in range(nchunks):
            for dsc in descs.pop(ci):
                dsc.wait()
            off, sz = chunks[ci]
            vv, rv, cv, _ = bufs[ci % 2]
            compute(vv, rv, cv, sz, UNROLL)
            if ci + 2 < nchunks:
                start(ci + 2)

        @pl.when(wid == 0)
        def _():
            d = pl.ds(0, TAIL_PAD)
            pltpu.sync_copy(tails_hbm.at[pl.ds(0, TAIL_PAD)], tvb)
            pltpu.sync_copy(tails_hbm.at[pl.ds(TAIL_PAD, TAIL_PAD)],
                            ra.at[d])
            pltpu.sync_copy(tails_hbm.at[pl.ds(2 * TAIL_PAD, TAIL_PAD)],
                            ca.at[d])

            def tbody(g, carry):
                prods = []
                for t in range(UNROLL):
                    j = g * UNROLL + t
                    s = pl.ds(j * L, L)
                    xg = plsc.load_gather(x_v, [ca[s]])
                    vv = plsc.bitcast(tvb[s], jnp.float32)
                    prods.append((ra[s], vv * xg))
                for rx, px in prods:
                    plsc.addupdate_scatter(acc_v, [rx], px)
                return carry

            lax.fori_loop(0, TAIL_PAD // (L * UNROLL), tbody, 0)

        pltpu.sync_copy(acc_v, out_hbm.at[wid])

    return k(vals, rows, cols, tails, x_flat)


def _finish(partials, b2, x2, Iy2, il2, iu2, l2, u2):
    def body(p_ref, b_ref, x_ref, iy_ref, il_ref, iu_ref, l_ref, u_ref,
             o_ref):
        ax = jnp.sum(p_ref[...], axis=0, keepdims=True)
        cons = ax - b_ref[...]
        cons = cons + jnp.maximum(-cons, 0.0) * iy_ref[...]
        xv = x_ref[...]
        var = (jnp.maximum(l_ref[...] - xv, 0.0) * il_ref[...]
               + jnp.maximum(xv - u_ref[...], 0.0) * iu_ref[...])
        part2 = jnp.maximum(jnp.max(jnp.abs(cons)), jnp.max(jnp.abs(var)))
        part3 = 1.0 + jnp.max(jnp.abs(b_ref[...]))
        o_ref[0, 0] = part2 / part3

    return pl.pallas_call(
        body,
        out_shape=jax.ShapeDtypeStruct((1, 1), jnp.float32),
        out_specs=pl.BlockSpec(memory_space=pltpu.SMEM),
    )(partials, b2, x2, Iy2, il2, iu2, l2, u2)


def kernel(A_vals, b, c, x, Iy, il, iu, l, u, A_rows, A_cols):
    nnz = A_vals.shape[0]
    n = x.shape[0]
    covered = (nnz // (NW * L)) * L * NW
    rows32 = A_rows.astype(jnp.int32)
    cols32 = A_cols.astype(jnp.int32)
    tails = jnp.pad(
        jnp.stack([
            lax.bitcast_convert_type(A_vals[covered:], jnp.int32),
            rows32[covered:],
            cols32[covered:],
        ]),
        ((0, 0), (0, TAIL_PAD - (nnz - covered))),
    ).reshape(-1)

    partials = _sc_partials(A_vals, rows32, cols32, tails, x[:, 0])

    r = lambda a: a.reshape(1, n)
    out = _finish(partials, b.reshape(1, -1), r(x), r(Iy), r(il), r(iu),
                  r(l), r(u))
    return out[0, 0]
